# baseline (speedup 1.0000x reference)
import jax
import jax.numpy as jnp
from jax.experimental import pallas as pl
from jax.experimental.pallas import tpu as pltpu


def _tail(h, dq_ref, b2_ref, w2_ref, emb_ref, zq_ref, ind_ref):
    bm = h.shape[0]
    h = jnp.where(h >= 0, h, 0.01 * h)
    z = jnp.dot(h, w2_ref[...]) + b2_ref[...]

    emb = emb_ref[...]
    ncodes = emb.shape[1]
    zsq = (z ** 2).sum(axis=1, keepdims=True)
    esq = (emb ** 2).sum(axis=0, keepdims=True)
    dist = zsq - 2.0 * jnp.dot(z, emb) + esq

    negd = -dist
    ind = jnp.argmax(negd, axis=1).astype(jnp.int32)
    minval = -jnp.max(negd, axis=1, keepdims=True)
    iota = jax.lax.broadcasted_iota(jnp.int32, (bm, ncodes), 1)

    onehot = (iota == ind[:, None]).astype(jnp.float32)
    q = jax.lax.dot_general(onehot, emb, (((1,), (1,)), ((), ())))

    dq = dq_ref[0] != 0
    zq_ref[...] = jnp.where(dq, q, z)
    ind_ref[...] = ind.reshape(1, bm)
    return jnp.sum(minval)


def _body(dq_ref, x_ref, w1_ref, b1_ref, b2_ref, w2_hbm, emb_hbm,
          zq_ref, ind_ref, diff_ref,
          hacc_ref, w2_ref, emb_ref, acc_ref, sem_w2, sem_emb):
    k = pl.program_id(0)
    nk = pl.num_programs(0) - 1
    b = hacc_ref.shape[0]
    half = b // 2
    dm = w2_ref.shape[1]

    w2_copy = pltpu.make_async_copy(w2_hbm, w2_ref, sem_w2)
    emb_copy = pltpu.make_async_copy(emb_hbm, emb_ref, sem_emb)

    @pl.when(k == 0)
    def _start():
        w2_copy.start()
        emb_copy.start()
        hacc_ref[...] = jnp.dot(x_ref[...], w1_ref[...])

    @pl.when((k > 0) & (k < nk - 1))
    def _mid():
        hacc_ref[...] += jnp.dot(x_ref[...], w1_ref[...])

    @pl.when(k == nk - 1)
    def _tail0():
        w2_copy.wait()
        emb_copy.wait()
        partial = jnp.dot(x_ref[...], w1_ref[...])
        hacc_ref[pl.ds(half, half), :] += partial[half:, :]
        h = (hacc_ref[pl.ds(0, half), :] + partial[:half, :]) + b1_ref[...]
        acc_ref[0] = _tail(h, dq_ref, b2_ref, w2_ref, emb_ref,
                           zq_ref, ind_ref)

    @pl.when(k == nk)
    def _tail1():
        h = hacc_ref[pl.ds(half, half), :] + b1_ref[...]
        s = _tail(h, dq_ref, b2_ref, w2_ref, emb_ref, zq_ref, ind_ref)
        dq = dq_ref[0] != 0
        diff_ref[0, 0] = jnp.where(dq, (acc_ref[0] + s) / (b * dm), 0.0)


def _encode(dq, x, w1, b1, b2, w2, emb, *, kc=1024, interpret=False):
    b, inp = x.shape
    dh = w1.shape[1]
    dm, ncodes = emb.shape
    nk = inp // kc
    half = b // 2
    zq, ind, diff = pl.pallas_call(
        _body,
        grid=(nk + 1,),
        in_specs=[
            pl.BlockSpec(memory_space=pltpu.SMEM),
            pl.BlockSpec((b, kc), lambda k: (0, jnp.minimum(k, nk - 1))),
            pl.BlockSpec((kc, dh), lambda k: (jnp.minimum(k, nk - 1), 0)),
            pl.BlockSpec((1, dh), lambda k: (0, 0)),
            pl.BlockSpec((1, dm), lambda k: (0, 0)),
            pl.BlockSpec(memory_space=pl.ANY),
            pl.BlockSpec(memory_space=pl.ANY),
        ],
        out_specs=[
            pl.BlockSpec((half, dm),
                         lambda k: (jnp.maximum(k - (nk - 1), 0), 0)),
            pl.BlockSpec((1, half),
                         lambda k: (0, jnp.maximum(k - (nk - 1), 0))),
            pl.BlockSpec(memory_space=pltpu.SMEM),
        ],
        out_shape=[
            jax.ShapeDtypeStruct((b, dm), jnp.float32),
            jax.ShapeDtypeStruct((1, b), jnp.int32),
            jax.ShapeDtypeStruct((1, 1), jnp.float32),
        ],
        scratch_shapes=[
            pltpu.VMEM((b, dh), jnp.float32),
            pltpu.VMEM((dh, dm), jnp.float32),
            pltpu.VMEM((dm, ncodes), jnp.float32),
            pltpu.SMEM((1,), jnp.float32),
            pltpu.SemaphoreType.DMA,
            pltpu.SemaphoreType.DMA,
        ],
        compiler_params=pltpu.CompilerParams(
            dimension_semantics=("arbitrary",),
        ),
        interpret=interpret,
    )(dq, x, w1, b1, b2, w2, emb)
    return zq, ind, diff


def kernel(x, W1, b1, W2, b2, embed, do_quantize, k):
    b = x.shape[0]
    xin = x.reshape((b, -1))
    dq = jnp.asarray(do_quantize, jnp.int32).reshape(1)
    zq, ind, diff = _encode(
        dq, xin, W1, b1.reshape(1, -1), b2.reshape(1, -1), W2, embed)
    return zq, diff.reshape(()), ind


# speedup vs baseline: 1.0274x; 1.0274x over previous
import jax
import jax.numpy as jnp
from jax.experimental import pallas as pl
from jax.experimental.pallas import tpu as pltpu


def _body(dq_ref, x_ref, w1_ref, b1_ref, b2_ref, w2_hbm, emb_hbm,
          zq_ref, ind_ref, diff_ref,
          hacc_ref, w2_ref, emb_ref, sem_w2, sem_emb):
    k = pl.program_id(0)
    nk = pl.num_programs(0)
    bm = x_ref.shape[0]

    w2_copy = pltpu.make_async_copy(w2_hbm, w2_ref, sem_w2)
    emb_copy = pltpu.make_async_copy(emb_hbm, emb_ref, sem_emb)

    @pl.when(k == 0)
    def _start():
        w2_copy.start()
        emb_copy.start()

    partial = jnp.dot(x_ref[...], w1_ref[...])

    @pl.when(k == 0)
    def _first():
        hacc_ref[...] = partial

    @pl.when((k > 0) & (k < nk - 1))
    def _rest():
        hacc_ref[...] += partial

    @pl.when(k == nk - 1)
    def _tail():
        w2_copy.wait()
        emb_copy.wait()

        h = (hacc_ref[...] + partial) + b1_ref[...]
        h = jnp.where(h >= 0, h, 0.01 * h)
        z = jnp.dot(h, w2_ref[...]) + b2_ref[...]

        emb = emb_ref[...]
        ncodes = emb.shape[1]
        dm = emb.shape[0]
        zsq = (z ** 2).sum(axis=1, keepdims=True)
        esq = (emb ** 2).sum(axis=0, keepdims=True)
        dist = zsq - 2.0 * jnp.dot(z, emb) + esq

        negd = -dist
        ind = jnp.argmax(negd, axis=1).astype(jnp.int32)
        minval = -jnp.max(negd, axis=1, keepdims=True)
        iota = jax.lax.broadcasted_iota(jnp.int32, (bm, ncodes), 1)

        onehot = (iota == ind[:, None]).astype(jnp.float32)
        q = jax.lax.dot_general(onehot, emb, (((1,), (1,)), ((), ())))

        dq = dq_ref[0] != 0
        zq_ref[...] = jnp.where(dq, q, z)
        ind_ref[...] = ind.reshape(1, bm)
        diff_ref[0, 0] = jnp.where(dq, jnp.sum(minval) / (bm * dm), 0.0)


def _encode(dq, x, w1, b1, b2, w2, emb, *, kc=1024, interpret=False):
    b, inp = x.shape
    dh = w1.shape[1]
    dm, ncodes = emb.shape
    nk = inp // kc
    zq, ind, diff = pl.pallas_call(
        _body,
        grid=(nk,),
        in_specs=[
            pl.BlockSpec(memory_space=pltpu.SMEM),
            pl.BlockSpec((b, kc), lambda k: (0, k)),
            pl.BlockSpec((kc, dh), lambda k: (k, 0)),
            pl.BlockSpec((1, dh), lambda k: (0, 0)),
            pl.BlockSpec((1, dm), lambda k: (0, 0)),
            pl.BlockSpec(memory_space=pl.ANY),
            pl.BlockSpec(memory_space=pl.ANY),
        ],
        out_specs=[
            pl.BlockSpec((b, dm), lambda k: (0, 0)),
            pl.BlockSpec((1, b), lambda k: (0, 0)),
            pl.BlockSpec(memory_space=pltpu.SMEM),
        ],
        out_shape=[
            jax.ShapeDtypeStruct((b, dm), jnp.float32),
            jax.ShapeDtypeStruct((1, b), jnp.int32),
            jax.ShapeDtypeStruct((1, 1), jnp.float32),
        ],
        scratch_shapes=[
            pltpu.VMEM((b, dh), jnp.float32),
            pltpu.VMEM((dh, dm), jnp.float32),
            pltpu.VMEM((dm, ncodes), jnp.float32),
            pltpu.SemaphoreType.DMA,
            pltpu.SemaphoreType.DMA,
        ],
        compiler_params=pltpu.CompilerParams(
            dimension_semantics=("arbitrary",),
        ),
        interpret=interpret,
    )(dq, x, w1, b1, b2, w2, emb)
    return zq, ind, diff


def kernel(x, W1, b1, W2, b2, embed, do_quantize, k):
    b = x.shape[0]
    xin = x.reshape((b, -1))
    dq = jnp.asarray(do_quantize, jnp.int32).reshape(1)
    zq, ind, diff = _encode(
        dq, xin, W1, b1.reshape(1, -1), b2.reshape(1, -1), W2, embed)
    return zq, diff.reshape(()), ind
